# baseline (speedup 1.0000x reference)
import functools

import jax
import jax.numpy as jnp
from jax import lax
from jax.experimental import pallas as pl
from jax.experimental.pallas import tpu as pltpu
from jax.experimental.pallas import tpu_sc as plsc

E = 8
D = 2048
DFF = 4096
S = 2048
CAP = 256
TB = 256
NBLK = S // TB
ZROW = S


def _gate_body(x_ref, wg_ref, out_ref, cnt_ref, acc_ref):
    i = pl.program_id(0)

    @pl.when(i == 0)
    def _init():
        cnt_ref[...] = jnp.zeros_like(cnt_ref)
        acc_ref[...] = jnp.zeros_like(acc_ref)

    x = x_ref[...]
    wg = wg_ref[...]
    logits = jnp.dot(x, wg, preferred_element_type=jnp.float32)
    rowmax = jnp.max(logits, axis=1, keepdims=True)
    iota_e = lax.broadcasted_iota(jnp.int32, (TB, E), 1).astype(jnp.float32)
    first_e = jnp.min(jnp.where(logits >= rowmax, iota_e, float(E)),
                      axis=1, keepdims=True)
    onehot = (iota_e == first_e).astype(jnp.float32)

    tri = (lax.broadcasted_iota(jnp.int32, (TB, TB), 0) >=
           lax.broadcasted_iota(jnp.int32, (TB, TB), 1)).astype(jnp.float32)
    prefix = jnp.dot(tri, onehot, preferred_element_type=jnp.float32)
    loc_all = prefix - 1.0 + cnt_ref[...]
    cnt_ref[...] += jnp.sum(onehot, axis=0, keepdims=True)

    loc_s = jnp.sum(loc_all * onehot, axis=1, keepdims=True)
    keep = loc_s < float(CAP)
    slot = first_e * float(CAP) + loc_s

    iota_j = lax.broadcasted_iota(jnp.int32, (TB, E * CAP), 1
                                  ).astype(jnp.float32)
    onehot_slot = jnp.where(keep & (slot == iota_j), 1.0, 0.0)
    svals = (jnp.float32(i * TB + 1) +
             lax.broadcasted_iota(jnp.int32, (1, TB), 1).astype(jnp.float32))
    acc_ref[...] += jnp.dot(svals, onehot_slot,
                            preferred_element_type=jnp.float32,
                            precision=lax.Precision.HIGHEST)

    @pl.when(i == pl.num_programs(0) - 1)
    def _fin():
        a = acc_ref[...]
        out_ref[...] = jnp.where(a > 0.5, a - 1.0,
                                 float(ZROW)).astype(jnp.int32)


def _gate(x, wg):
    return pl.pallas_call(
        _gate_body,
        grid=(NBLK,),
        in_specs=[
            pl.BlockSpec((TB, D), lambda i: (i, 0)),
            pl.BlockSpec((D, E), lambda i: (0, 0)),
        ],
        out_specs=pl.BlockSpec((1, E * CAP), lambda i: (0, 0)),
        out_shape=jax.ShapeDtypeStruct((1, E * CAP), jnp.int32),
        scratch_shapes=[
            pltpu.VMEM((1, E), jnp.float32),
            pltpu.VMEM((1, E * CAP), jnp.float32),
        ],
        compiler_params=pltpu.CompilerParams(
            dimension_semantics=("arbitrary",)),
    )(x, wg)


_NC, _NS = 2, 16
_NW = _NC * _NS
_RPW = (E * CAP) // _NW
_CHUNK = 32
_NCH = _RPW // _CHUNK


@functools.cache
def _make_gather():
    mesh = plsc.VectorSubcoreMesh(core_axis_name="c", subcore_axis_name="s",
                                  num_cores=_NC, num_subcores=_NS)

    @functools.partial(
        pl.kernel, mesh=mesh,
        out_type=jax.ShapeDtypeStruct((E * CAP, D), jnp.float32),
        scratch_types=[
            pltpu.VMEM((_NCH, _CHUNK), jnp.int32),
            pltpu.VMEM((_CHUNK, D), jnp.float32),
            pltpu.SemaphoreType.DMA,
        ],
    )
    def gather_k(table_hbm, idx_hbm, out_hbm, idx_v, rows_v, sem):
        wid = lax.axis_index("s") * _NC + lax.axis_index("c")
        base = wid * _RPW
        pltpu.sync_copy(idx_hbm.at[wid], idx_v)
        for c in range(_NCH):
            pltpu.async_copy(table_hbm.at[idx_v.at[c]], rows_v, sem).wait()
            pltpu.sync_copy(rows_v, out_hbm.at[pl.ds(base + c * _CHUNK,
                                                     _CHUNK)])

    return gather_k


NT = 512


def _ffn_body(xd_ref, w_ref, b_ref, out_ref):
    out_ref[...] = (jnp.dot(xd_ref[...], w_ref[0],
                            preferred_element_type=jnp.float32)
                    + b_ref[0])


def _ffn(xd, W, b):
    return pl.pallas_call(
        _ffn_body,
        grid=(E, DFF // NT),
        in_specs=[
            pl.BlockSpec((CAP, D), lambda e, n: (e, 0)),
            pl.BlockSpec((1, D, NT), lambda e, n: (e, 0, n)),
            pl.BlockSpec((1, 1, NT), lambda e, n: (e, 0, n)),
        ],
        out_specs=pl.BlockSpec((CAP, NT), lambda e, n: (e, n)),
        out_shape=jax.ShapeDtypeStruct((E * CAP, DFF), jnp.float32),
        compiler_params=pltpu.CompilerParams(
            dimension_semantics=("parallel", "parallel")),
    )(xd, W, b.reshape(E, 1, DFF))


def kernel(inputs, wg, W, b):
    x2 = inputs.reshape(-1, D)
    gidx = _gate(x2, wg)
    gidx3 = gidx.reshape(_NW, _NCH, _CHUNK)
    xz = jnp.concatenate([x2, jnp.zeros((8, D), jnp.float32)], axis=0)
    disp = _make_gather()(xz, gidx3)
    return _ffn(disp, W, b)


# speedup vs baseline: 1.1350x; 1.1350x over previous
import functools

import jax
import jax.numpy as jnp
from jax import lax
from jax.experimental import pallas as pl
from jax.experimental.pallas import tpu as pltpu
from jax.experimental.pallas import tpu_sc as plsc

E = 8
D = 2048
DFF = 4096
S = 2048
CAP = 256
TB = 256
NBLK = S // TB


def _gate_body(x_ref, wg_ref, idx_ref, val_ref, cnt_ref, acc_ref):
    i = pl.program_id(0)

    @pl.when(i == 0)
    def _init():
        cnt_ref[...] = jnp.zeros_like(cnt_ref)
        acc_ref[...] = jnp.zeros_like(acc_ref)

    x = x_ref[...]
    logits = jnp.dot(x, wg_ref[...], preferred_element_type=jnp.float32)
    rowmax = jnp.max(logits, axis=1, keepdims=True)
    iota_e = lax.broadcasted_iota(jnp.int32, (TB, E), 1).astype(jnp.float32)
    first_e = jnp.min(jnp.where(logits >= rowmax, iota_e, float(E)),
                      axis=1, keepdims=True)
    onehot = (iota_e == first_e).astype(jnp.float32)

    tri = (lax.broadcasted_iota(jnp.int32, (TB, TB), 0) >=
           lax.broadcasted_iota(jnp.int32, (TB, TB), 1)).astype(jnp.float32)
    prefix = jnp.dot(tri, onehot, preferred_element_type=jnp.float32)
    loc_all = prefix - 1.0 + cnt_ref[...]
    cnt_ref[...] += jnp.sum(onehot, axis=0, keepdims=True)

    loc_s = jnp.sum(loc_all * onehot, axis=1, keepdims=True)
    keep = loc_s < float(CAP)

    iota_c = lax.broadcasted_iota(jnp.int32, (TB, CAP), 1).astype(jnp.float32)
    onehot_c = jnp.where(keep & (loc_s == iota_c), 1.0, 0.0)
    sv = (jnp.float32(i * TB + 1) +
          lax.broadcasted_iota(jnp.int32, (TB, 1), 0).astype(jnp.float32))
    acc_ref[...] += lax.dot_general(
        onehot * sv, onehot_c, (((0,), (0,)), ((), ())),
        preferred_element_type=jnp.float32,
        precision=lax.Precision.HIGHEST)

    @pl.when(i == pl.num_programs(0) - 1)
    def _fin():
        a = acc_ref[...]
        idx_ref[...] = jnp.where(a > 0.5, a - 1.0, 0.0).astype(jnp.int32)
        val_ref[...] = (a > 0.5).astype(jnp.float32)


def _gate(x, wg):
    return pl.pallas_call(
        _gate_body,
        grid=(NBLK,),
        in_specs=[
            pl.BlockSpec((TB, D), lambda i: (i, 0)),
            pl.BlockSpec((D, E), lambda i: (0, 0)),
        ],
        out_specs=[
            pl.BlockSpec((E, CAP), lambda i: (0, 0)),
            pl.BlockSpec((E, CAP), lambda i: (0, 0)),
        ],
        out_shape=[
            jax.ShapeDtypeStruct((E, CAP), jnp.int32),
            jax.ShapeDtypeStruct((E, CAP), jnp.float32),
        ],
        scratch_shapes=[
            pltpu.VMEM((1, E), jnp.float32),
            pltpu.VMEM((E, CAP), jnp.float32),
        ],
        compiler_params=pltpu.CompilerParams(
            dimension_semantics=("arbitrary",)),
    )(x, wg)


_NC, _NS = 2, 16
_NW = _NC * _NS
_RPW = (E * CAP) // _NW
_CHUNK = 16
_NCH = _RPW // _CHUNK


@functools.cache
def _make_gather():
    mesh = plsc.VectorSubcoreMesh(core_axis_name="c", subcore_axis_name="s",
                                  num_cores=_NC, num_subcores=_NS)

    @functools.partial(
        pl.kernel, mesh=mesh,
        out_type=jax.ShapeDtypeStruct((E * CAP, D), jnp.float32),
        scratch_types=[
            pltpu.VMEM((_NCH, _CHUNK), jnp.int32),
            pltpu.VMEM((_CHUNK, D), jnp.float32),
            pltpu.VMEM((_CHUNK, D), jnp.float32),
            pltpu.SemaphoreType.DMA,
            pltpu.SemaphoreType.DMA,
        ],
    )
    def gather_k(table_hbm, idx_hbm, out_hbm, idx_v, rows_a, rows_b, sem_a,
                 sem_b):
        wid = lax.axis_index("s") * _NC + lax.axis_index("c")
        base = wid * _RPW
        pltpu.sync_copy(idx_hbm.at[wid], idx_v)
        bufs = (rows_a, rows_b)
        sems = (sem_a, sem_b)
        cps = [pltpu.async_copy(table_hbm.at[idx_v.at[0]], rows_a, sem_a)]
        for c in range(_NCH):
            if c + 1 < _NCH:
                cps.append(pltpu.async_copy(
                    table_hbm.at[idx_v.at[c + 1]],
                    bufs[(c + 1) % 2], sems[(c + 1) % 2]))
            cps[c].wait()
            pltpu.sync_copy(bufs[c % 2],
                            out_hbm.at[pl.ds(base + c * _CHUNK, _CHUNK)])

    return gather_k


NT = 512


def _ffn_body(xd_ref, w_ref, m_ref, b_ref, out_ref):
    acc = jnp.dot(xd_ref[...].astype(jnp.bfloat16),
                  w_ref[0].astype(jnp.bfloat16),
                  preferred_element_type=jnp.float32)
    out_ref[...] = m_ref[0] * acc + b_ref[0]


def _ffn(xd, W, mask, b):
    return pl.pallas_call(
        _ffn_body,
        grid=(E, DFF // NT),
        in_specs=[
            pl.BlockSpec((CAP, D), lambda e, n: (e, 0)),
            pl.BlockSpec((1, D, NT), lambda e, n: (e, 0, n)),
            pl.BlockSpec((1, CAP, 1), lambda e, n: (e, 0, 0)),
            pl.BlockSpec((1, 1, NT), lambda e, n: (e, 0, n)),
        ],
        out_specs=pl.BlockSpec((CAP, NT), lambda e, n: (e, n)),
        out_shape=jax.ShapeDtypeStruct((E * CAP, DFF), jnp.float32),
        compiler_params=pltpu.CompilerParams(
            dimension_semantics=("parallel", "parallel")),
    )(xd, W, mask.reshape(E, CAP, 1), b.reshape(E, 1, DFF))


def kernel(inputs, wg, W, b):
    x2 = inputs.reshape(-1, D)
    gidx, valid = _gate(x2, wg)
    disp = _make_gather()(x2, gidx.reshape(_NW, _NCH, _CHUNK))
    return _ffn(disp, W, valid, b)


# speedup vs baseline: 1.2501x; 1.1014x over previous
import functools

import jax
import jax.numpy as jnp
from jax import lax
from jax.experimental import pallas as pl
from jax.experimental.pallas import tpu as pltpu
from jax.experimental.pallas import tpu_sc as plsc

E = 8
D = 2048
DFF = 4096
S = 2048
CAP = 256
TB = 256
NBLK = S // TB
EH = E // 2


def _gate_body(x_ref, wg_ref, idx_ref, val_ref, cnt_ref, acc_ref):
    i = pl.program_id(0)

    @pl.when(i == 0)
    def _init():
        cnt_ref[...] = jnp.zeros_like(cnt_ref)
        acc_ref[...] = jnp.zeros_like(acc_ref)

    x = x_ref[...]
    logits = jnp.dot(x, wg_ref[...], preferred_element_type=jnp.float32)
    rowmax = jnp.max(logits, axis=1, keepdims=True)
    iota_e = lax.broadcasted_iota(jnp.int32, (TB, E), 1).astype(jnp.float32)
    first_e = jnp.min(jnp.where(logits >= rowmax, iota_e, float(E)),
                      axis=1, keepdims=True)
    onehot = (iota_e == first_e).astype(jnp.float32)

    tri = (lax.broadcasted_iota(jnp.int32, (TB, TB), 0) >=
           lax.broadcasted_iota(jnp.int32, (TB, TB), 1)).astype(jnp.float32)
    prefix = jnp.dot(tri, onehot, preferred_element_type=jnp.float32)
    loc_all = prefix - 1.0 + cnt_ref[...]
    cnt_ref[...] += jnp.sum(onehot, axis=0, keepdims=True)

    loc_s = jnp.sum(loc_all * onehot, axis=1, keepdims=True)
    keep = loc_s < float(CAP)

    iota_c = lax.broadcasted_iota(jnp.int32, (TB, CAP), 1).astype(jnp.float32)
    onehot_c = jnp.where(keep & (loc_s == iota_c), 1.0, 0.0)
    sv = (jnp.float32(i * TB + 1) +
          lax.broadcasted_iota(jnp.int32, (TB, 1), 0).astype(jnp.float32))
    acc_ref[...] += lax.dot_general(
        onehot * sv, onehot_c, (((0,), (0,)), ((), ())),
        preferred_element_type=jnp.float32,
        precision=lax.Precision.HIGHEST)

    @pl.when(i == pl.num_programs(0) - 1)
    def _fin():
        a = acc_ref[...]
        idx_ref[...] = jnp.where(a > 0.5, a - 1.0, 0.0).astype(jnp.int32)
        val_ref[...] = (a > 0.5).astype(jnp.float32)


def _gate(x, wg):
    return pl.pallas_call(
        _gate_body,
        grid=(NBLK,),
        in_specs=[
            pl.BlockSpec((TB, D), lambda i: (i, 0)),
            pl.BlockSpec((D, E), lambda i: (0, 0)),
        ],
        out_specs=[
            pl.BlockSpec((E, CAP), lambda i: (0, 0)),
            pl.BlockSpec((E, CAP), lambda i: (0, 0)),
        ],
        out_shape=[
            jax.ShapeDtypeStruct((E, CAP), jnp.int32),
            jax.ShapeDtypeStruct((E, CAP), jnp.float32),
        ],
        scratch_shapes=[
            pltpu.VMEM((1, E), jnp.float32),
            pltpu.VMEM((E, CAP), jnp.float32),
        ],
        compiler_params=pltpu.CompilerParams(
            dimension_semantics=("arbitrary",)),
    )(x, wg)


_NC, _NS = 2, 16
_NW = _NC * _NS
_ROWS = EH * CAP
_RPW = _ROWS // _NW
_CHUNK = 16
_NCH = _RPW // _CHUNK


@functools.cache
def _make_gather():
    mesh = plsc.VectorSubcoreMesh(core_axis_name="c", subcore_axis_name="s",
                                  num_cores=_NC, num_subcores=_NS)

    @functools.partial(
        pl.kernel, mesh=mesh,
        out_type=jax.ShapeDtypeStruct((_ROWS, D), jnp.float32),
        scratch_types=[
            pltpu.VMEM((_NCH, _CHUNK), jnp.int32),
            pltpu.VMEM((_CHUNK, D), jnp.float32),
            pltpu.VMEM((_CHUNK, D), jnp.float32),
            pltpu.SemaphoreType.DMA,
            pltpu.SemaphoreType.DMA,
        ],
    )
    def gather_k(table_hbm, idx_hbm, out_hbm, idx_v, rows_a, rows_b, sem_a,
                 sem_b):
        wid = lax.axis_index("s") * _NC + lax.axis_index("c")
        base = wid * _RPW
        pltpu.sync_copy(idx_hbm.at[wid], idx_v)
        bufs = (rows_a, rows_b)
        sems = (sem_a, sem_b)
        cps = [pltpu.async_copy(table_hbm.at[idx_v.at[0]], rows_a, sem_a)]
        for c in range(_NCH):
            if c + 1 < _NCH:
                cps.append(pltpu.async_copy(
                    table_hbm.at[idx_v.at[c + 1]],
                    bufs[(c + 1) % 2], sems[(c + 1) % 2]))
            cps[c].wait()
            pltpu.sync_copy(bufs[c % 2],
                            out_hbm.at[pl.ds(base + c * _CHUNK, _CHUNK)])

    return gather_k


NT = 1024


def _ffn_body(xd_ref, w_ref, m_ref, b_ref, out_ref):
    acc = jnp.dot(xd_ref[...].astype(jnp.bfloat16),
                  w_ref[0].astype(jnp.bfloat16),
                  preferred_element_type=jnp.float32)
    out_ref[...] = m_ref[0] * acc + b_ref[0]


def _ffn_alias_body(prev_ref, xd_ref, w_ref, m_ref, b_ref, out_ref):
    del prev_ref
    _ffn_body(xd_ref, w_ref, m_ref, b_ref, out_ref)


def _ffn_half(xd, W, mask, b, off, prev=None):
    in_specs = [
        pl.BlockSpec((CAP, D), lambda e, n: (e, 0)),
        pl.BlockSpec((1, D, NT), lambda e, n: (e + off, 0, n)),
        pl.BlockSpec((1, CAP, 1), lambda e, n: (e + off, 0, 0)),
        pl.BlockSpec((1, 1, NT), lambda e, n: (e + off, 0, n)),
    ]
    args = [xd, W, mask, b]
    body = _ffn_body
    kwargs = {}
    if prev is not None:
        in_specs = [pl.BlockSpec(memory_space=pl.ANY)] + in_specs
        args = [prev] + args
        body = _ffn_alias_body
        kwargs["input_output_aliases"] = {0: 0}
    return pl.pallas_call(
        body,
        grid=(EH, DFF // NT),
        in_specs=in_specs,
        out_specs=pl.BlockSpec((CAP, NT), lambda e, n: (e + off, n)),
        out_shape=jax.ShapeDtypeStruct((E * CAP, DFF), jnp.float32),
        compiler_params=pltpu.CompilerParams(
            dimension_semantics=("parallel", "parallel")),
        **kwargs,
    )(*args)


def kernel(inputs, wg, W, b):
    x2 = inputs.reshape(-1, D)
    gidx, valid = _gate(x2, wg)
    idx3 = gidx.reshape(2, _NW, _NCH, _CHUNK)
    mask3 = valid.reshape(E, CAP, 1)
    b3 = b.reshape(E, 1, DFF)
    gather = _make_gather()
    disp_a = gather(x2, idx3[0])
    disp_b = gather(x2, idx3[1])
    out_a = _ffn_half(disp_a, W, mask3, b3, 0)
    return _ffn_half(disp_b, W, mask3, b3, EH, prev=out_a)
